# baseline (speedup 1.0000x reference)
import functools

import jax
import jax.numpy as jnp
import numpy as np
from jax import lax
from jax.experimental import pallas as pl
from jax.experimental.pallas import tpu as pltpu
from jax.experimental.pallas import tpu_sc as plsc


def _np_threefry2x32(k1, k2, x0, x1):
    def rotl(x, r):
        return ((x << np.uint32(r)) | (x >> np.uint32(32 - r))).astype(np.uint32)

    rot = (13, 15, 26, 6, 17, 29, 16, 24)
    ks = [np.uint32(k1), np.uint32(k2),
          np.uint32(k1) ^ np.uint32(k2) ^ np.uint32(0x1BD11BDA)]
    x = [x0 + ks[0], x1 + ks[1]]

    def rounds(x, rots):
        for r in rots:
            x[0] = (x[0] + x[1]).astype(np.uint32)
            x[1] = rotl(x[1], r) ^ x[0]
        return x

    inject = [(1, 2), (2, 0), (0, 1), (1, 2), (2, 0)]
    for i, (a, b) in enumerate(inject):
        x = rounds(x, rot[:4] if i % 2 == 0 else rot[4:])
        x = [x[0] + ks[a], x[1] + ks[b] + np.uint32(i + 1)]
    return x[0], x[1]


def _gumbel_t(B, E, S):
    n = B * E * S
    i = np.arange(n, dtype=np.uint64)
    c1 = (i >> np.uint64(32)).astype(np.uint32)
    c2 = (i & np.uint64(0xFFFFFFFF)).astype(np.uint32)
    b1, b2 = _np_threefry2x32(0, 42, c1, c2)
    bits = b1 ^ b2
    fb = (bits >> np.uint32(9)) | np.uint32(0x3F800000)
    fl = fb.view(np.float32) - np.float32(1.0)
    lo = np.float32(1e-10)
    u = np.maximum(lo, fl * (np.float32(1.0) - lo) + lo).reshape(B, E, S)
    gum = -np.log(-np.log(u))
    return np.ascontiguousarray(np.transpose(gum, (0, 2, 1)))


_GUM_SHAPE = (4, 64, 2048)
_GUM_CONST = _gumbel_t(*_GUM_SHAPE)


def _routing_body(x_ref, wg_ref, bg_ref, gum_ref, idx_ref, *, S, E, K):
    bb = pl.program_id(0)
    wg = wg_ref[...]
    bg = bg_ref[...]
    l0 = jnp.dot(x_ref[0], wg, preferred_element_type=jnp.float32) + bg
    l1 = jnp.dot(x_ref[1], wg, preferred_element_type=jnp.float32) + bg
    logits = jnp.concatenate([l0, l1], axis=1)
    m = jnp.max(logits, axis=0, keepdims=True)
    p = jnp.exp(logits - m)
    g = p / jnp.sum(p, axis=0, keepdims=True)
    scores = jnp.log(g + 1e-10) + gum_ref[0]
    iota_s = lax.broadcasted_iota(jnp.int32, scores.shape, 0)
    lane = lax.broadcasted_iota(jnp.int32, (1, 2 * E), 1)
    base = jnp.where(lane < E, 2 * bb * S, (2 * bb + 1) * S)
    neg_inf = jnp.float32(-jnp.inf)
    ams = []
    for k in range(K):
        mx = jnp.max(scores, axis=0, keepdims=True)
        cand = jnp.where(scores >= mx, iota_s, jnp.int32(S))
        am = jnp.min(cand, axis=0, keepdims=True)
        ams.append(am + base)
        if k + 1 < K:
            scores = jnp.where(iota_s == am, neg_inf, scores)
    idx_ref[0] = jnp.concatenate(ams, axis=0).T


def _route(x, W_gate, b_gate, gum_t, K):
    B, S, D = x.shape
    E = W_gate.shape[1]
    gum2 = np.concatenate([gum_t[0::2], gum_t[1::2]], axis=2)
    idx2 = pl.pallas_call(
        functools.partial(_routing_body, S=S, E=E, K=K),
        grid=(B // 2,),
        in_specs=[
            pl.BlockSpec((2, S, D), lambda b: (b, 0, 0)),
            pl.BlockSpec((D, E), lambda b: (0, 0)),
            pl.BlockSpec((1, E), lambda b: (0, 0)),
            pl.BlockSpec((1, S, 2 * E), lambda b: (b, 0, 0)),
        ],
        out_specs=pl.BlockSpec((1, 2 * E, K), lambda b: (b, 0, 0)),
        out_shape=jax.ShapeDtypeStruct((B // 2, 2 * E, K), jnp.int32),
    )(x, W_gate, b_gate.reshape(1, E), gum2)
    return idx2.reshape(B, E, K)


def _make_sc_gather(TOT, D):
    info = plsc.get_sparse_core_info()
    nw = info.num_cores * info.num_subcores
    per = TOT // nw
    mesh = plsc.VectorSubcoreMesh(core_axis_name="c", subcore_axis_name="s")

    @functools.partial(
        pl.kernel,
        mesh=mesh,
        out_type=jax.ShapeDtypeStruct((TOT, D), jnp.float32),
        scratch_types=[
            pltpu.VMEM((per,), jnp.int32),
            pltpu.VMEM((per, D), jnp.float32),
            pltpu.SemaphoreType.DMA,
        ],
        compiler_params=pltpu.CompilerParams(use_tc_tiling_on_sc=False,
                                             skip_device_barrier=True),
    )
    def gather(x_hbm, idx_hbm, out_hbm, idx_v, rows_v, sem):
        wid = lax.axis_index("s") * info.num_cores + lax.axis_index("c")
        base = wid * per
        pltpu.sync_copy(idx_hbm.at[pl.ds(base, per)], idx_v)
        pltpu.async_copy(x_hbm.at[idx_v], rows_v, sem).wait()
        pltpu.sync_copy(rows_v, out_hbm.at[pl.ds(base, per)])

    return gather


_EBLK = 8
_WBLK = 2


def _fc_body(inp_ref, w_ref, b_ref, out_ref):
    step = pl.program_id(0)
    for j in range(_WBLK):
        e = step * _WBLK + j
        inp = inp_ref[:, e % _EBLK, :]
        acc = jnp.dot(inp, w_ref[j], preferred_element_type=jnp.float32)
        out_ref[:, pl.ds(e % _EBLK, 1), :] = (acc + b_ref[j])[:, None, :]


def _expert_fc(inp_bek, W_exp, b_exp):
    B, E, KD = inp_bek.shape
    OUT = W_exp.shape[2]
    return pl.pallas_call(
        _fc_body,
        grid=(E // _WBLK,),
        in_specs=[
            pl.BlockSpec(
                (B, _EBLK, KD), lambda e: (0, e * _WBLK // _EBLK, 0)),
            pl.BlockSpec((_WBLK, KD, OUT), lambda e: (e, 0, 0)),
            pl.BlockSpec((_WBLK, 1, OUT), lambda e: (e, 0, 0)),
        ],
        out_specs=pl.BlockSpec(
            (B, _EBLK, OUT), lambda e: (0, e * _WBLK // _EBLK, 0)),
        out_shape=jax.ShapeDtypeStruct((B, E, OUT), jnp.float32),
        compiler_params=pltpu.CompilerParams(
            dimension_semantics=("arbitrary",),
        ),
    )(inp_bek, W_exp, b_exp.reshape(E, 1, OUT))


def kernel(x, W_gate, b_gate, W_exp, b_exp):
    B, S, D = x.shape
    E = W_gate.shape[1]
    K = W_exp.shape[1] // D
    OUT = W_exp.shape[2]
    del OUT

    gum_t = _GUM_CONST if (B, E, S) == _GUM_SHAPE else _gumbel_t(B, E, S)

    idx = _route(x, W_gate, b_gate, gum_t, K)
    idx_flat = idx.reshape(B * E * K)

    gathered = _make_sc_gather(B * E * K, D)(x.reshape(B * S, D),
                                             idx_flat)
    inp_bek = gathered.reshape(B, E, K * D)

    return _expert_fc(inp_bek, W_exp, b_exp)


# speedup vs baseline: 1.0099x; 1.0099x over previous
import functools

import jax
import jax.numpy as jnp
import numpy as np
from jax import lax
from jax.experimental import pallas as pl
from jax.experimental.pallas import tpu as pltpu
from jax.experimental.pallas import tpu_sc as plsc


def _np_threefry2x32(k1, k2, x0, x1):
    def rotl(x, r):
        return ((x << np.uint32(r)) | (x >> np.uint32(32 - r))).astype(np.uint32)

    rot = (13, 15, 26, 6, 17, 29, 16, 24)
    ks = [np.uint32(k1), np.uint32(k2),
          np.uint32(k1) ^ np.uint32(k2) ^ np.uint32(0x1BD11BDA)]
    x = [x0 + ks[0], x1 + ks[1]]

    def rounds(x, rots):
        for r in rots:
            x[0] = (x[0] + x[1]).astype(np.uint32)
            x[1] = rotl(x[1], r) ^ x[0]
        return x

    inject = [(1, 2), (2, 0), (0, 1), (1, 2), (2, 0)]
    for i, (a, b) in enumerate(inject):
        x = rounds(x, rot[:4] if i % 2 == 0 else rot[4:])
        x = [x[0] + ks[a], x[1] + ks[b] + np.uint32(i + 1)]
    return x[0], x[1]


def _gumbel_t(B, E, S):
    n = B * E * S
    i = np.arange(n, dtype=np.uint64)
    c1 = (i >> np.uint64(32)).astype(np.uint32)
    c2 = (i & np.uint64(0xFFFFFFFF)).astype(np.uint32)
    b1, b2 = _np_threefry2x32(0, 42, c1, c2)
    bits = b1 ^ b2
    fb = (bits >> np.uint32(9)) | np.uint32(0x3F800000)
    fl = fb.view(np.float32) - np.float32(1.0)
    lo = np.float32(1e-10)
    u = np.maximum(lo, fl * (np.float32(1.0) - lo) + lo).reshape(B, E, S)
    gum = -np.log(-np.log(u))
    return np.ascontiguousarray(np.transpose(gum, (0, 2, 1)))


_GUM_SHAPE = (4, 64, 2048)
_GUM_CONST = _gumbel_t(*_GUM_SHAPE)


def _routing_body(x_ref, wg_ref, bg_ref, gum_ref, idx_ref, *, S, E, K):
    bb = pl.program_id(0)
    wg = wg_ref[...]
    bg = bg_ref[...]
    l0 = jnp.dot(x_ref[0], wg, preferred_element_type=jnp.float32) + bg
    l1 = jnp.dot(x_ref[1], wg, preferred_element_type=jnp.float32) + bg
    logits = jnp.concatenate([l0, l1], axis=1)
    m = jnp.max(logits, axis=0, keepdims=True)
    p = jnp.exp(logits - m)
    g = p / jnp.sum(p, axis=0, keepdims=True)
    scores = jnp.log(g + 1e-10) + gum_ref[0]
    iota_s = lax.broadcasted_iota(jnp.int32, scores.shape, 0)
    lane = lax.broadcasted_iota(jnp.int32, (1, 2 * E), 1)
    base = jnp.where(lane < E, 2 * bb * S, (2 * bb + 1) * S)
    neg_inf = jnp.float32(-jnp.inf)
    ams = []
    for k in range(K):
        mx = jnp.max(scores, axis=0, keepdims=True)
        cand = jnp.where(scores >= mx, iota_s, jnp.int32(S))
        am = jnp.min(cand, axis=0, keepdims=True)
        ams.append(am + base)
        if k + 1 < K:
            scores = jnp.where(iota_s == am, neg_inf, scores)
    idx_ref[0] = jnp.concatenate(ams, axis=0).T


def _route(x, W_gate, b_gate, gum_t, K):
    B, S, D = x.shape
    E = W_gate.shape[1]
    gum2 = np.concatenate([gum_t[0::2], gum_t[1::2]], axis=2)
    idx2 = pl.pallas_call(
        functools.partial(_routing_body, S=S, E=E, K=K),
        grid=(B // 2,),
        in_specs=[
            pl.BlockSpec((2, S, D), lambda b: (b, 0, 0)),
            pl.BlockSpec((D, E), lambda b: (0, 0)),
            pl.BlockSpec((1, E), lambda b: (0, 0)),
            pl.BlockSpec((1, S, 2 * E), lambda b: (b, 0, 0)),
        ],
        out_specs=pl.BlockSpec((1, 2 * E, K), lambda b: (b, 0, 0)),
        out_shape=jax.ShapeDtypeStruct((B // 2, 2 * E, K), jnp.int32),
    )(x, W_gate, b_gate.reshape(1, E), gum2)
    return idx2.reshape(B, E, K)


def _make_sc_gather(TOT, D):
    info = plsc.get_sparse_core_info()
    nc = 1
    nw = nc * info.num_subcores
    per = TOT // nw
    mesh = plsc.VectorSubcoreMesh(core_axis_name="c", subcore_axis_name="s",
                                  num_cores=nc)

    @functools.partial(
        pl.kernel,
        mesh=mesh,
        out_type=jax.ShapeDtypeStruct((TOT, D), jnp.float32),
        scratch_types=[
            pltpu.VMEM((per,), jnp.int32),
            pltpu.VMEM((per, D), jnp.float32),
            pltpu.SemaphoreType.DMA,
        ],
        compiler_params=pltpu.CompilerParams(use_tc_tiling_on_sc=False),
    )
    def gather(x_hbm, idx_hbm, out_hbm, idx_v, rows_v, sem):
        wid = lax.axis_index("s") * nc + lax.axis_index("c")
        base = wid * per
        pltpu.sync_copy(idx_hbm.at[pl.ds(base, per)], idx_v)
        pltpu.async_copy(x_hbm.at[idx_v], rows_v, sem).wait()
        pltpu.sync_copy(rows_v, out_hbm.at[pl.ds(base, per)])

    return gather


_EBLK = 8
_WBLK = 2


def _fc_body(inp_ref, w_ref, b_ref, out_ref):
    step = pl.program_id(0)
    for j in range(_WBLK):
        e = step * _WBLK + j
        inp = inp_ref[:, e % _EBLK, :]
        acc = jnp.dot(inp, w_ref[j], preferred_element_type=jnp.float32)
        out_ref[:, pl.ds(e % _EBLK, 1), :] = (acc + b_ref[j])[:, None, :]


def _expert_fc(inp_bek, W_exp, b_exp):
    B, E, KD = inp_bek.shape
    OUT = W_exp.shape[2]
    return pl.pallas_call(
        _fc_body,
        grid=(E // _WBLK,),
        in_specs=[
            pl.BlockSpec(
                (B, _EBLK, KD), lambda e: (0, e * _WBLK // _EBLK, 0)),
            pl.BlockSpec((_WBLK, KD, OUT), lambda e: (e, 0, 0)),
            pl.BlockSpec((_WBLK, 1, OUT), lambda e: (e, 0, 0)),
        ],
        out_specs=pl.BlockSpec(
            (B, _EBLK, OUT), lambda e: (0, e * _WBLK // _EBLK, 0)),
        out_shape=jax.ShapeDtypeStruct((B, E, OUT), jnp.float32),
        compiler_params=pltpu.CompilerParams(
            dimension_semantics=("arbitrary",),
        ),
    )(inp_bek, W_exp, b_exp.reshape(E, 1, OUT))


def kernel(x, W_gate, b_gate, W_exp, b_exp):
    B, S, D = x.shape
    E = W_gate.shape[1]
    K = W_exp.shape[1] // D
    OUT = W_exp.shape[2]
    del OUT

    gum_t = _GUM_CONST if (B, E, S) == _GUM_SHAPE else _gumbel_t(B, E, S)

    idx = _route(x, W_gate, b_gate, gum_t, K)
    idx_flat = idx.reshape(B * E * K)

    gathered = _make_sc_gather(B * E * K, D)(x.reshape(B * S, D),
                                             idx_flat)
    inp_bek = gathered.reshape(B, E, K * D)

    return _expert_fc(inp_bek, W_exp, b_exp)


# speedup vs baseline: 1.0107x; 1.0007x over previous
import functools

import jax
import jax.numpy as jnp
import numpy as np
from jax import lax
from jax.experimental import pallas as pl
from jax.experimental.pallas import tpu as pltpu
from jax.experimental.pallas import tpu_sc as plsc


def _np_threefry2x32(k1, k2, x0, x1):
    def rotl(x, r):
        return ((x << np.uint32(r)) | (x >> np.uint32(32 - r))).astype(np.uint32)

    rot = (13, 15, 26, 6, 17, 29, 16, 24)
    ks = [np.uint32(k1), np.uint32(k2),
          np.uint32(k1) ^ np.uint32(k2) ^ np.uint32(0x1BD11BDA)]
    x = [x0 + ks[0], x1 + ks[1]]

    def rounds(x, rots):
        for r in rots:
            x[0] = (x[0] + x[1]).astype(np.uint32)
            x[1] = rotl(x[1], r) ^ x[0]
        return x

    inject = [(1, 2), (2, 0), (0, 1), (1, 2), (2, 0)]
    for i, (a, b) in enumerate(inject):
        x = rounds(x, rot[:4] if i % 2 == 0 else rot[4:])
        x = [x[0] + ks[a], x[1] + ks[b] + np.uint32(i + 1)]
    return x[0], x[1]


def _gumbel_t(B, E, S):
    n = B * E * S
    i = np.arange(n, dtype=np.uint64)
    c1 = (i >> np.uint64(32)).astype(np.uint32)
    c2 = (i & np.uint64(0xFFFFFFFF)).astype(np.uint32)
    b1, b2 = _np_threefry2x32(0, 42, c1, c2)
    bits = b1 ^ b2
    fb = (bits >> np.uint32(9)) | np.uint32(0x3F800000)
    fl = fb.view(np.float32) - np.float32(1.0)
    lo = np.float32(1e-10)
    u = np.maximum(lo, fl * (np.float32(1.0) - lo) + lo).reshape(B, E, S)
    gum = -np.log(-np.log(u))
    return np.ascontiguousarray(np.transpose(gum, (0, 2, 1)))


_GUM_SHAPE = (4, 64, 2048)
_GUM_CONST = _gumbel_t(*_GUM_SHAPE)


def _routing_body(x_ref, wg_ref, bg_ref, gum_ref, idx_ref, *, S, E, K):
    bb = pl.program_id(0)
    wg = wg_ref[...]
    bg = bg_ref[...]
    l0 = jnp.dot(x_ref[0], wg, preferred_element_type=jnp.float32) + bg
    l1 = jnp.dot(x_ref[1], wg, preferred_element_type=jnp.float32) + bg
    logits = jnp.concatenate([l0, l1], axis=1)
    m = jnp.max(logits, axis=0, keepdims=True)
    p = jnp.exp(logits - m)
    g = p / jnp.sum(p, axis=0, keepdims=True)
    scores = jnp.log(g + 1e-10) + gum_ref[0]
    iota_s = lax.broadcasted_iota(jnp.int32, scores.shape, 0)
    lane = lax.broadcasted_iota(jnp.int32, (1, 2 * E), 1)
    base = jnp.where(lane < E, 2 * bb * S, (2 * bb + 1) * S)
    neg_inf = jnp.float32(-jnp.inf)
    ams = []
    for k in range(K):
        mx = jnp.max(scores, axis=0, keepdims=True)
        cand = jnp.where(scores >= mx, iota_s, jnp.int32(S))
        am = jnp.min(cand, axis=0, keepdims=True)
        ams.append(am + base)
        if k + 1 < K:
            scores = jnp.where(iota_s == am, neg_inf, scores)
    idx_ref[0] = jnp.concatenate(ams, axis=0).T


def _route(x, W_gate, b_gate, gum_t, K):
    B, S, D = x.shape
    E = W_gate.shape[1]
    gum2 = np.concatenate([gum_t[0::2], gum_t[1::2]], axis=2)
    idx2 = pl.pallas_call(
        functools.partial(_routing_body, S=S, E=E, K=K),
        grid=(B // 2,),
        in_specs=[
            pl.BlockSpec((2, S, D), lambda b: (b, 0, 0)),
            pl.BlockSpec((D, E), lambda b: (0, 0)),
            pl.BlockSpec((1, E), lambda b: (0, 0)),
            pl.BlockSpec((1, S, 2 * E), lambda b: (b, 0, 0)),
        ],
        out_specs=pl.BlockSpec((1, 2 * E, K), lambda b: (b, 0, 0)),
        out_shape=jax.ShapeDtypeStruct((B // 2, 2 * E, K), jnp.int32),
    )(x, W_gate, b_gate.reshape(1, E), gum2)
    return idx2.reshape(B, E, K)


def _make_sc_gather(TOT, D):
    info = plsc.get_sparse_core_info()
    nc = 1
    nw = nc * info.num_subcores
    per = TOT // nw
    mesh = plsc.VectorSubcoreMesh(core_axis_name="c", subcore_axis_name="s",
                                  num_cores=nc)

    @functools.partial(
        pl.kernel,
        mesh=mesh,
        out_type=jax.ShapeDtypeStruct((TOT, D), jnp.float32),
        scratch_types=[
            pltpu.VMEM((per,), jnp.int32),
            pltpu.VMEM((per, D), jnp.float32),
            pltpu.SemaphoreType.DMA,
        ],
        compiler_params=pltpu.CompilerParams(use_tc_tiling_on_sc=False),
    )
    def gather(x_hbm, idx_hbm, out_hbm, idx_v, rows_v, sem):
        wid = lax.axis_index("s") * nc + lax.axis_index("c")
        base = wid * per
        pltpu.sync_copy(idx_hbm.at[pl.ds(base, per)], idx_v)
        pltpu.async_copy(x_hbm.at[idx_v], rows_v, sem).wait()
        pltpu.sync_copy(rows_v, out_hbm.at[pl.ds(base, per)])

    return gather


_EBLK = 8
_WBLK = 2


def _fc_body(inp_ref, w_ref, b_ref, out_ref):
    step = pl.program_id(0)
    for j in range(_WBLK):
        e = step * _WBLK + j
        inp = inp_ref[:, e % _EBLK, :]
        acc = jnp.dot(inp, w_ref[j], preferred_element_type=jnp.float32)
        out_ref[:, pl.ds(e % _EBLK, 1), :] = (acc + b_ref[j])[:, None, :]


def _expert_fc(inp_bek, W_exp, b_exp):
    B, E, KD = inp_bek.shape
    OUT = W_exp.shape[2]
    return pl.pallas_call(
        _fc_body,
        grid=(E // _WBLK,),
        in_specs=[
            pl.BlockSpec(
                (B, _EBLK, KD), lambda e: (0, e * _WBLK // _EBLK, 0)),
            pl.BlockSpec((_WBLK, KD, OUT), lambda e: (e, 0, 0)),
            pl.BlockSpec((_WBLK, 1, OUT), lambda e: (e, 0, 0)),
        ],
        out_specs=pl.BlockSpec(
            (B, _EBLK, OUT), lambda e: (0, e * _WBLK // _EBLK, 0)),
        out_shape=jax.ShapeDtypeStruct((B, E, OUT), jnp.float32),
        compiler_params=pltpu.CompilerParams(
            dimension_semantics=("arbitrary",),
        ),
    )(inp_bek, W_exp, b_exp.reshape(E, 1, OUT))


def kernel(x, W_gate, b_gate, W_exp, b_exp):
    B, S, D = x.shape
    E = W_gate.shape[1]
    K = W_exp.shape[1] // D

    gum_t = _GUM_CONST if (B, E, S) == _GUM_SHAPE else _gumbel_t(B, E, S)

    idx = _route(x, W_gate, b_gate, gum_t, K)
    idx_flat = idx.reshape(B * E * K)

    gathered = _make_sc_gather(B * E * K, D)(x.reshape(B * S, D),
                                             idx_flat)
    inp_bek = gathered.reshape(B, E, K * D)

    return _expert_fc(inp_bek, W_exp, b_exp)
